# baseline (speedup 1.0000x reference)
import functools

import jax
import jax.numpy as jnp
from jax import lax
from jax.experimental import pallas as pl
from jax.experimental.pallas import tpu as pltpu
from jax.experimental.pallas import tpu_sc as plsc

_Q = 0.7
_K = 0.5

_B = 16384
_C = 1000
_BLKB = 1024
_NB = _B // _BLKB

_NW = 32
_BPW = _B // _NW
_CHUNK = 128
_NCH = _BPW // _CHUNK

_C1 = 1.0 / _Q - (1.0 - _K ** _Q) / _Q


def _gather_weights_sc(weight_flat, indexes):
    mesh = plsc.VectorSubcoreMesh(core_axis_name="c", subcore_axis_name="s")

    @functools.partial(
        pl.kernel,
        mesh=mesh,
        out_type=jax.ShapeDtypeStruct((_B,), jnp.float32),
        scratch_types=[
            pltpu.VMEM((_BPW,), jnp.int32),
            pltpu.VMEM((_BPW,), jnp.float32),
            pltpu.SemaphoreType.DMA,
        ],
    )
    def gather_kernel(table_hbm, idx_hbm, out_hbm, idx_v, rows_v, sem):
        wid = lax.axis_index("s") * 2 + lax.axis_index("c")
        base = wid * _BPW
        pltpu.sync_copy(idx_hbm.at[pl.ds(base, _BPW)], idx_v)
        copies = [
            pltpu.async_copy(
                table_hbm.at[idx_v.at[pl.ds(j * _CHUNK, _CHUNK)]],
                rows_v.at[pl.ds(j * _CHUNK, _CHUNK)],
                sem,
            )
            for j in range(_NCH)
        ]
        for c in copies:
            c.wait()
        pltpu.sync_copy(rows_v, out_hbm.at[pl.ds(base, _BPW)])

    return gather_kernel(weight_flat, indexes)


def _ygq_body(lt_ref, targets_ref, ygq_ref):
    l = lt_ref[...]
    t = targets_ref[...]
    row = lax.broadcasted_iota(jnp.int32, (_C, _BLKB), 0)
    m = jnp.max(l, axis=0, keepdims=True)
    e = jnp.exp(l - m)
    s = jnp.sum(e, axis=0, keepdims=True)
    lt = jnp.sum(jnp.where(row == t, l, 0.0), axis=0, keepdims=True)
    yg = jnp.exp(lt - m) / s
    ygq_ref[...] = jnp.exp(_Q * jnp.log(yg))


def _combine_body(ygq_ref, w_ref, out_ref):
    out_ref[...] = jnp.sum(
        (_C1 - ygq_ref[...] * (1.0 / _Q)) * w_ref[...],
        keepdims=True,
    ) * (1.0 / _B)


def kernel(logits, targets, indexes, weight):
    ygq = pl.pallas_call(
        _ygq_body,
        grid=(_NB,),
        in_specs=[
            pl.BlockSpec((_C, _BLKB), lambda i: (0, i)),
            pl.BlockSpec((1, _BLKB), lambda i: (0, i)),
        ],
        out_specs=pl.BlockSpec((1, _BLKB), lambda i: (0, i)),
        out_shape=jax.ShapeDtypeStruct((1, _B), jnp.float32),
    )(logits.T, targets.astype(jnp.int32).reshape(1, _B))
    w = _gather_weights_sc(weight.reshape(-1), indexes.astype(jnp.int32))
    out = pl.pallas_call(
        _combine_body,
        in_specs=[
            pl.BlockSpec((1, _B), lambda: (0, 0)),
            pl.BlockSpec((1, _B), lambda: (0, 0)),
        ],
        out_specs=pl.BlockSpec((1, 1), lambda: (0, 0)),
        out_shape=jax.ShapeDtypeStruct((1, 1), jnp.float32),
    )(ygq, w.reshape(1, _B))
    return out[0, 0]


# speedup vs baseline: 1.0062x; 1.0062x over previous
import functools

import jax
import jax.numpy as jnp
from jax import lax
from jax.experimental import pallas as pl
from jax.experimental.pallas import tpu as pltpu
from jax.experimental.pallas import tpu_sc as plsc

_Q = 0.7
_K = 0.5

_B = 16384
_C = 1000
_BLKB = 2048
_NB = _B // _BLKB

_NW = 32
_BPW = _B // _NW
_CHUNK = 128
_NCH = _BPW // _CHUNK

_C1 = 1.0 / _Q - (1.0 - _K ** _Q) / _Q


def _gather_weights_sc(weight_flat, indexes):
    mesh = plsc.VectorSubcoreMesh(core_axis_name="c", subcore_axis_name="s")

    @functools.partial(
        pl.kernel,
        mesh=mesh,
        out_type=jax.ShapeDtypeStruct((_B,), jnp.float32),
        scratch_types=[
            pltpu.VMEM((_BPW,), jnp.int32),
            pltpu.VMEM((_BPW,), jnp.float32),
            pltpu.SemaphoreType.DMA,
        ],
    )
    def gather_kernel(table_hbm, idx_hbm, out_hbm, idx_v, rows_v, sem):
        wid = lax.axis_index("s") * 2 + lax.axis_index("c")
        base = wid * _BPW
        pltpu.sync_copy(idx_hbm.at[pl.ds(base, _BPW)], idx_v)
        copies = [
            pltpu.async_copy(
                table_hbm.at[idx_v.at[pl.ds(j * _CHUNK, _CHUNK)]],
                rows_v.at[pl.ds(j * _CHUNK, _CHUNK)],
                sem,
            )
            for j in range(_NCH)
        ]
        for c in copies:
            c.wait()
        pltpu.sync_copy(rows_v, out_hbm.at[pl.ds(base, _BPW)])

    return gather_kernel(weight_flat, indexes)


def _ygq_body(lt_ref, targets_ref, ygq_ref):
    l = lt_ref[...]
    t = targets_ref[...]
    row = lax.broadcasted_iota(jnp.int32, (_C, _BLKB), 0)
    m = jnp.max(l, axis=0, keepdims=True)
    e = jnp.exp(l - m)
    s = jnp.sum(e, axis=0, keepdims=True)
    lt = jnp.sum(jnp.where(row == t, l, 0.0), axis=0, keepdims=True)
    yg = jnp.exp(lt - m) / s
    ygq_ref[...] = jnp.exp(_Q * jnp.log(yg))


def _combine_body(ygq_ref, w_ref, out_ref):
    out_ref[...] = jnp.sum(
        (_C1 - ygq_ref[...] * (1.0 / _Q)) * w_ref[...],
        keepdims=True,
    ) * (1.0 / _B)


def kernel(logits, targets, indexes, weight):
    ygq = pl.pallas_call(
        _ygq_body,
        grid=(_NB,),
        in_specs=[
            pl.BlockSpec((_C, _BLKB), lambda i: (0, i)),
            pl.BlockSpec((1, _BLKB), lambda i: (0, i)),
        ],
        out_specs=pl.BlockSpec((1, _BLKB), lambda i: (0, i)),
        out_shape=jax.ShapeDtypeStruct((1, _B), jnp.float32),
    )(logits.T, targets.astype(jnp.int32).reshape(1, _B))
    idx_b, ygq = lax.optimization_barrier(
        (indexes.astype(jnp.int32), ygq)
    )
    w = _gather_weights_sc(weight.reshape(-1), idx_b)
    out = pl.pallas_call(
        _combine_body,
        in_specs=[
            pl.BlockSpec((1, _B), lambda: (0, 0)),
            pl.BlockSpec((1, _B), lambda: (0, 0)),
        ],
        out_specs=pl.BlockSpec((1, 1), lambda: (0, 0)),
        out_shape=jax.ShapeDtypeStruct((1, 1), jnp.float32),
    )(ygq, w.reshape(1, _B))
    return out[0, 0]


# speedup vs baseline: 1.0509x; 1.0445x over previous
import functools

import jax
import jax.numpy as jnp
from jax import lax
from jax.experimental import pallas as pl
from jax.experimental.pallas import tpu as pltpu
from jax.experimental.pallas import tpu_sc as plsc

_Q = 0.7
_K = 0.5

_B = 16384
_C = 1000
_BLKB = 2048
_NB = _B // _BLKB

_NW = 16
_BPW = _B // _NW
_CHUNK = 128
_NCH = _BPW // _CHUNK

_C1 = 1.0 / _Q - (1.0 - _K ** _Q) / _Q


def _gather_weights_sc(weight_flat, indexes):
    mesh = plsc.VectorSubcoreMesh(
        core_axis_name="c", subcore_axis_name="s", num_cores=1
    )

    @functools.partial(
        pl.kernel,
        mesh=mesh,
        out_type=jax.ShapeDtypeStruct((_B,), jnp.float32),
        scratch_types=[
            pltpu.VMEM((_BPW,), jnp.int32),
            pltpu.VMEM((_BPW,), jnp.float32),
            pltpu.SemaphoreType.DMA,
        ],
    )
    def gather_kernel(table_hbm, idx_hbm, out_hbm, idx_v, rows_v, sem):
        wid = lax.axis_index("s") * 1 + lax.axis_index("c")
        base = wid * _BPW
        pltpu.sync_copy(idx_hbm.at[pl.ds(base, _BPW)], idx_v)
        copies = [
            pltpu.async_copy(
                table_hbm.at[idx_v.at[pl.ds(j * _CHUNK, _CHUNK)]],
                rows_v.at[pl.ds(j * _CHUNK, _CHUNK)],
                sem,
            )
            for j in range(_NCH)
        ]
        for c in copies:
            c.wait()
        pltpu.sync_copy(rows_v, out_hbm.at[pl.ds(base, _BPW)])

    return gather_kernel(weight_flat, indexes)


def _ygq_body(lt_ref, targets_ref, ygq_ref):
    l = lt_ref[...]
    t = targets_ref[...]
    row = lax.broadcasted_iota(jnp.int32, (_C, _BLKB), 0)
    m = jnp.max(l, axis=0, keepdims=True)
    e = jnp.exp(l - m)
    s = jnp.sum(e, axis=0, keepdims=True)
    lt = jnp.sum(jnp.where(row == t, l, 0.0), axis=0, keepdims=True)
    yg = jnp.exp(lt - m) / s
    ygq_ref[...] = jnp.exp(_Q * jnp.log(yg))


def _combine_body(ygq_ref, w_ref, out_ref):
    out_ref[...] = jnp.sum(
        (_C1 - ygq_ref[...] * (1.0 / _Q)) * w_ref[...],
        keepdims=True,
    ) * (1.0 / _B)


def kernel(logits, targets, indexes, weight):
    ygq = pl.pallas_call(
        _ygq_body,
        grid=(_NB,),
        in_specs=[
            pl.BlockSpec((_C, _BLKB), lambda i: (0, i)),
            pl.BlockSpec((1, _BLKB), lambda i: (0, i)),
        ],
        out_specs=pl.BlockSpec((1, _BLKB), lambda i: (0, i)),
        out_shape=jax.ShapeDtypeStruct((1, _B), jnp.float32),
    )(logits.T, targets.astype(jnp.int32).reshape(1, _B))
    w = _gather_weights_sc(weight.reshape(-1), indexes.astype(jnp.int32))
    out = pl.pallas_call(
        _combine_body,
        in_specs=[
            pl.BlockSpec((1, _B), lambda: (0, 0)),
            pl.BlockSpec((1, _B), lambda: (0, 0)),
        ],
        out_specs=pl.BlockSpec((1, 1), lambda: (0, 0)),
        out_shape=jax.ShapeDtypeStruct((1, 1), jnp.float32),
    )(ygq, w.reshape(1, _B))
    return out[0, 0]
